# baseline (speedup 1.0000x reference)
import jax
import jax.numpy as jnp
from jax.experimental import pallas as pl
from jax.experimental.pallas import tpu as pltpu

_BS = 512


def _add_body(feat_ref, pos_ref, out_ref):
    out_ref[...] = feat_ref[...] + pos_ref[...][None, :, :]


def _half(feat, pe, b0, nb):
    B, S, D = feat.shape
    return pl.pallas_call(
        _add_body,
        grid=(S // _BS,),
        in_specs=[
            pl.BlockSpec((nb, _BS, D), lambda i, b0=b0: (b0, i, 0)),
            pl.BlockSpec((_BS, D), lambda i: (i, 0)),
        ],
        out_specs=pl.BlockSpec((nb, _BS, D), lambda i: (0, i, 0)),
        out_shape=jax.ShapeDtypeStruct((nb, S, D), feat.dtype),
        compiler_params=pltpu.CompilerParams(
            dimension_semantics=("parallel",),
        ),
    )(feat, pe)


def kernel(feat, pos_emb):
    B, S, D = feat.shape
    pe = pos_emb[:S]
    lo = _half(feat, pe, 0, 2)
    hi = _half(feat, pe, 1, 2)
    return jnp.concatenate([lo, hi], axis=0)
